# baseline (speedup 1.0000x reference)
import functools

import jax
import jax.numpy as jnp
from jax import lax
from jax.experimental import pallas as pl
from jax.experimental.pallas import tpu as pltpu
from jax.experimental.pallas import tpu_sc as plsc

N = 10000
NPAD = 10240
F = 128
D3 = 64
NCLS = 40
E = 320000
C = 128
EPAD = 327680
KCH = EPAD // (32 * C)
HN = NPAD // 2
ORS = HN // 16

_MESH = dict(core_axis_name="c", subcore_axis_name="s")

CW = 64
NB = 4
NPH = 4


def _make_edge_kernel(with_gather):
    HK = EPAD // (32 * CW * NPH)

    @functools.partial(
        pl.kernel,
        out_type=jax.ShapeDtypeStruct((2 * NPAD, F), jnp.float32),
        mesh=plsc.VectorSubcoreMesh(**_MESH),
        scratch_types=[
            pltpu.VMEM((HK, 2 * CW), jnp.int32),
            pltpu.VMEM((1, CW), jnp.int32),
            pltpu.VMEM((1, CW), jnp.int32),
            [pltpu.VMEM((CW, F), jnp.float32) for _ in range(NB)],
            pltpu.VMEM_SHARED((HN, F), jnp.float32),
            pltpu.VMEM_SHARED((HN, F), jnp.float32),
            [pltpu.SemaphoreType.DMA for _ in range(NB)],
        ],
    )
    def edge_kernel(*refs):
        (g_hbm, ed_hbm, zeros_hbm, out_hbm, ed_v,
         dsta_v, dstb_v, rows, acca, accb, sems) = refs
        cid = lax.axis_index("c")
        sid = lax.axis_index("s")
        wid = cid * 16 + sid

        def split_scatter(j, rows_v):
            for k in range(CW // 16):
                d = ed_v[j, pl.ds(CW + k * 16, 16)]
                lo = d < HN
                dsta_v[0, pl.ds(k * 16, 16)] = jnp.where(lo, d, -1)
                dstb_v[0, pl.ds(k * 16, 16)] = jnp.where(lo, -1, d - HN)
            pltpu.sync_copy(
                rows_v,
                acca.at[plsc.Indices(dsta_v.at[0], ignored_value=-1)],
                add=True)
            pltpu.sync_copy(
                rows_v,
                accb.at[plsc.Indices(dstb_v.at[0], ignored_value=-1)],
                add=True)

        if not with_gather:
            pltpu.sync_copy(g_hbm.at[pl.ds(0, CW)], rows[0])
        pltpu.sync_copy(zeros_hbm, acca.at[pl.ds(sid * ORS, ORS)])
        pltpu.sync_copy(zeros_hbm, accb.at[pl.ds(sid * ORS, ORS)])
        plsc.subcore_barrier()

        for ph in range(NPH):
            base = wid * NPH * HK + ph * HK
            pltpu.sync_copy(ed_hbm.at[pl.ds(base, HK)], ed_v)

            if with_gather:
                def gidx(j):
                    return ed_v.at[j, pl.ds(0, CW)]

                for t in range(NB):
                    pltpu.async_copy(g_hbm.at[gidx(t)], rows[t], sems[t])

                def body(jj, carry):
                    for t in range(NB):
                        j = NB * jj + t
                        pltpu.make_async_copy(
                            g_hbm.at[gidx(j)], rows[t], sems[t]).wait()
                        split_scatter(j, rows[t])

                        @pl.when(j + NB < HK)
                        def _prefetch():
                            pltpu.async_copy(
                                g_hbm.at[gidx(j + NB)], rows[t], sems[t])
                    return carry
                lax.fori_loop(0, HK // NB, body, 0)
            else:
                def body(j, carry):
                    split_scatter(j, rows[0])
                    return carry
                lax.fori_loop(0, HK, body, 0)

        plsc.subcore_barrier()
        pltpu.sync_copy(acca.at[pl.ds(sid * ORS, ORS)],
                        out_hbm.at[pl.ds(cid * NPAD + sid * ORS, ORS)])
        pltpu.sync_copy(accb.at[pl.ds(sid * ORS, ORS)],
                        out_hbm.at[pl.ds(cid * NPAD + HN + sid * ORS, ORS)])

    return edge_kernel


_scatter = _make_edge_kernel(True)
_count = _make_edge_kernel(False)


_BLK = 512
_GRID = NPAD // _BLK
_ROW = pl.BlockSpec((_BLK, F), lambda i: (i, 0))
_COL1 = pl.BlockSpec((_BLK, 1), lambda i: (i, 0))


def _dot(a, b):
    return lax.dot_general(a, b, (((1,), (0,)), ((), ())),
                           precision=lax.Precision.HIGHEST,
                           preferred_element_type=jnp.float32)


def _tcmm_body(x_ref, w_ref, h_ref):
    h_ref[...] = _dot(x_ref[...], w_ref[...])


def _tc_matmul(x, w):
    return pl.pallas_call(
        _tcmm_body,
        grid=(_GRID,),
        in_specs=[_ROW, pl.BlockSpec((F, F), lambda i: (0, 0))],
        out_specs=_ROW,
        out_shape=jax.ShapeDtypeStruct((NPAD, F), jnp.float32),
    )(x, w)


def _tc1_body(h_ref, c0_ref, c1_ref, dis_ref, g_ref):
    deg = 1.0 + c0_ref[...] + c1_ref[...]
    dis = lax.rsqrt(deg)
    dis_ref[...] = dis
    g_ref[...] = dis * h_ref[...]


def _tc_first(h, c0, c1):
    return pl.pallas_call(
        _tc1_body,
        grid=(_GRID,),
        in_specs=[_ROW, _COL1, _COL1],
        out_specs=[_COL1, _ROW],
        out_shape=[
            jax.ShapeDtypeStruct((NPAD, 1), jnp.float32),
            jax.ShapeDtypeStruct((NPAD, F), jnp.float32),
        ],
    )(h, c0, c1)


def _tcmid_body(matmul, s0, s1, g, dis_ref, b_ref, w_ref, out_ref):
    dis = dis_ref[...]
    h = dis * (s0[...] + s1[...] + g[...]) + b_ref[...]
    h = jnp.maximum(h, 0.0)
    if matmul:
        out_ref[...] = dis * _dot(h, w_ref[...])
    else:
        out_ref[...] = dis * h


def _tc_mid(s0, s1, g, dis, b, w, matmul):
    return pl.pallas_call(
        functools.partial(_tcmid_body, matmul),
        grid=(_GRID,),
        in_specs=[_ROW, _ROW, _ROW, _COL1,
                  pl.BlockSpec((1, F), lambda i: (0, 0)),
                  pl.BlockSpec((F, F), lambda i: (0, 0))],
        out_specs=_ROW,
        out_shape=jax.ShapeDtypeStruct((NPAD, F), jnp.float32),
    )(s0, s1, g, dis, b, w)


def _tc4_body(s0, s1, t_ref, dis_ref, w_ref, b_ref, out_ref):
    z = dis_ref[...] * (s0[...] + s1[...] + t_ref[...])
    logit = _dot(z, w_ref[...]) + b_ref[...]
    col = lax.broadcasted_iota(jnp.int32, logit.shape, 1)
    mask = col < NCLS
    logit = jnp.where(mask, logit, -jnp.inf)
    m = jnp.max(logit, axis=1, keepdims=True)
    e = jnp.where(mask, jnp.exp(logit - m), 0.0)
    out_ref[...] = e / jnp.sum(e, axis=1, keepdims=True)


def _tc_last(s0, s1, t, dis, w, b):
    return pl.pallas_call(
        _tc4_body,
        grid=(_GRID,),
        in_specs=[_ROW, _ROW, _ROW, _COL1,
                  pl.BlockSpec((F, D3), lambda i: (0, 0)),
                  pl.BlockSpec((1, D3), lambda i: (0, 0))],
        out_specs=pl.BlockSpec((_BLK, D3), lambda i: (i, 0)),
        out_shape=jax.ShapeDtypeStruct((NPAD, D3), jnp.float32),
    )(s0, s1, t, dis, w, b)


@jax.jit
def kernel(x, edge_index, W1, b1, W2, b2, W3, b3):
    ei = edge_index.astype(jnp.int32)
    pad = jnp.full((EPAD - E,), N, jnp.int32)
    src2d = jnp.concatenate([ei[0], pad]).reshape(EPAD // CW, CW)
    dst2d = jnp.concatenate([ei[1], pad]).reshape(EPAD // CW, CW)
    ed2d = jnp.concatenate([src2d, dst2d], axis=1)
    x_pad = jnp.pad(x, ((0, NPAD - N), (0, 0)))
    w3p = jnp.pad(W3, ((0, 0), (0, D3 - NCLS)))
    b1r = b1.reshape(1, F)
    b2r = b2.reshape(1, F)
    b3r = jnp.pad(b3, (0, D3 - NCLS)).reshape(1, D3)
    zeros = jnp.zeros((ORS, F), jnp.float32)
    ones = jnp.ones((CW, F), jnp.float32)

    h1 = _tc_matmul(x_pad, W1)
    cnt = _count(ones, ed2d, zeros)
    dis, g1 = _tc_first(h1, cnt[:NPAD, 0:1], cnt[NPAD:, 0:1])

    s1 = _scatter(g1, ed2d, zeros)
    g2 = _tc_mid(s1[:NPAD], s1[NPAD:], g1, dis, b1r, W2, matmul=True)

    s2 = _scatter(g2, ed2d, zeros)
    t3 = _tc_mid(s2[:NPAD], s2[NPAD:], g2, dis, b2r, W2, matmul=False)

    s3 = _scatter(t3, ed2d, zeros)
    probs = _tc_last(s3[:NPAD], s3[NPAD:], t3, dis, w3p, b3r)
    return probs[:N, :NCLS]


# speedup vs baseline: 1.0569x; 1.0569x over previous
import functools

import jax
import jax.numpy as jnp
from jax import lax
from jax.experimental import pallas as pl
from jax.experimental.pallas import tpu as pltpu
from jax.experimental.pallas import tpu_sc as plsc

N = 10000
NPAD = 10240
F = 128
D3 = 64
NCLS = 40
E = 320000
C = 128
EPAD = 327680
KCH = EPAD // (32 * C)
HN = NPAD // 2
ORS = HN // 16

_MESH = dict(core_axis_name="c", subcore_axis_name="s")

CW = 64
NB = 4
NPH = 4


def _make_edge_kernel(with_gather):
    HK = EPAD // (32 * CW * NPH)

    @functools.partial(
        pl.kernel,
        out_type=jax.ShapeDtypeStruct((2 * NPAD, F), jnp.float32),
        mesh=plsc.VectorSubcoreMesh(**_MESH),
        scratch_types=[
            pltpu.VMEM((HK, CW), jnp.int32),
            pltpu.VMEM((HK, CW), jnp.int32),
            pltpu.VMEM((1, CW), jnp.int32),
            pltpu.VMEM((1, CW), jnp.int32),
            [pltpu.VMEM((CW, F), jnp.float32) for _ in range(NB)],
            pltpu.VMEM_SHARED((HN, F), jnp.float32),
            pltpu.VMEM_SHARED((HN, F), jnp.float32),
            [pltpu.SemaphoreType.DMA for _ in range(NB)],
        ],
    )
    def edge_kernel(*refs):
        (g_hbm, src_hbm, dst_hbm, zeros_hbm, out_hbm, src_v, dst_v,
         dsta_v, dstb_v, rows, acca, accb, sems) = refs
        cid = lax.axis_index("c")
        sid = lax.axis_index("s")
        wid = cid * 16 + sid

        def split_scatter(j, rows_v):
            for k in range(CW // 16):
                d = dst_v[j, pl.ds(k * 16, 16)]
                lo = d < HN
                dsta_v[0, pl.ds(k * 16, 16)] = jnp.where(lo, d, -1)
                dstb_v[0, pl.ds(k * 16, 16)] = jnp.where(lo, -1, d - HN)
            pltpu.sync_copy(
                rows_v,
                acca.at[plsc.Indices(dsta_v.at[0], ignored_value=-1)],
                add=True)
            pltpu.sync_copy(
                rows_v,
                accb.at[plsc.Indices(dstb_v.at[0], ignored_value=-1)],
                add=True)

        if not with_gather:
            pltpu.sync_copy(g_hbm.at[pl.ds(0, CW)], rows[0])
        pltpu.sync_copy(zeros_hbm, acca.at[pl.ds(sid * ORS, ORS)])
        pltpu.sync_copy(zeros_hbm, accb.at[pl.ds(sid * ORS, ORS)])
        plsc.subcore_barrier()

        for ph in range(NPH):
            base = wid * NPH * HK + ph * HK
            if with_gather:
                pltpu.sync_copy(src_hbm.at[pl.ds(base, HK)], src_v)
            pltpu.sync_copy(dst_hbm.at[pl.ds(base, HK)], dst_v)

            if with_gather:
                def gidx(j):
                    return src_v.at[j]

                for t in range(NB):
                    pltpu.async_copy(g_hbm.at[gidx(t)], rows[t], sems[t])

                def body(jj, carry):
                    for t in range(NB):
                        j = NB * jj + t
                        pltpu.make_async_copy(
                            g_hbm.at[gidx(j)], rows[t], sems[t]).wait()
                        split_scatter(j, rows[t])

                        @pl.when(j + NB < HK)
                        def _prefetch():
                            pltpu.async_copy(
                                g_hbm.at[gidx(j + NB)], rows[t], sems[t])
                    return carry
                lax.fori_loop(0, HK // NB, body, 0)
            else:
                def body(j, carry):
                    split_scatter(j, rows[0])
                    return carry
                lax.fori_loop(0, HK, body, 0)

        plsc.subcore_barrier()
        pltpu.sync_copy(acca.at[pl.ds(sid * ORS, ORS)],
                        out_hbm.at[pl.ds(cid * NPAD + sid * ORS, ORS)])
        pltpu.sync_copy(accb.at[pl.ds(sid * ORS, ORS)],
                        out_hbm.at[pl.ds(cid * NPAD + HN + sid * ORS, ORS)])

    return edge_kernel


_scatter = _make_edge_kernel(True)
_count = _make_edge_kernel(False)


_BLK = 512
_GRID = NPAD // _BLK
_ROW = pl.BlockSpec((_BLK, F), lambda i: (i, 0))
_COL1 = pl.BlockSpec((_BLK, 1), lambda i: (i, 0))


def _dot(a, b):
    return lax.dot_general(a, b, (((1,), (0,)), ((), ())),
                           precision=lax.Precision.HIGHEST,
                           preferred_element_type=jnp.float32)


def _tcmm_body(x_ref, w_ref, h_ref):
    h_ref[...] = _dot(x_ref[...], w_ref[...])


def _tc_matmul(x, w):
    return pl.pallas_call(
        _tcmm_body,
        grid=(_GRID,),
        in_specs=[_ROW, pl.BlockSpec((F, F), lambda i: (0, 0))],
        out_specs=_ROW,
        out_shape=jax.ShapeDtypeStruct((NPAD, F), jnp.float32),
    )(x, w)


def _tc1_body(h_ref, c0_ref, c1_ref, dis_ref, g_ref):
    deg = 1.0 + c0_ref[...] + c1_ref[...]
    dis = lax.rsqrt(deg)
    dis_ref[...] = dis
    g_ref[...] = dis * h_ref[...]


def _tc_first(h, c0, c1):
    return pl.pallas_call(
        _tc1_body,
        grid=(_GRID,),
        in_specs=[_ROW, _COL1, _COL1],
        out_specs=[_COL1, _ROW],
        out_shape=[
            jax.ShapeDtypeStruct((NPAD, 1), jnp.float32),
            jax.ShapeDtypeStruct((NPAD, F), jnp.float32),
        ],
    )(h, c0, c1)


def _tcmid_body(matmul, s0, s1, g, dis_ref, b_ref, w_ref, out_ref):
    dis = dis_ref[...]
    h = dis * (s0[...] + s1[...] + g[...]) + b_ref[...]
    h = jnp.maximum(h, 0.0)
    if matmul:
        out_ref[...] = dis * _dot(h, w_ref[...])
    else:
        out_ref[...] = dis * h


def _tc_mid(s0, s1, g, dis, b, w, matmul):
    return pl.pallas_call(
        functools.partial(_tcmid_body, matmul),
        grid=(_GRID,),
        in_specs=[_ROW, _ROW, _ROW, _COL1,
                  pl.BlockSpec((1, F), lambda i: (0, 0)),
                  pl.BlockSpec((F, F), lambda i: (0, 0))],
        out_specs=_ROW,
        out_shape=jax.ShapeDtypeStruct((NPAD, F), jnp.float32),
    )(s0, s1, g, dis, b, w)


def _tc4_body(s0, s1, t_ref, dis_ref, w_ref, b_ref, out_ref):
    z = dis_ref[...] * (s0[...] + s1[...] + t_ref[...])
    logit = _dot(z, w_ref[...]) + b_ref[...]
    col = lax.broadcasted_iota(jnp.int32, logit.shape, 1)
    mask = col < NCLS
    logit = jnp.where(mask, logit, -jnp.inf)
    m = jnp.max(logit, axis=1, keepdims=True)
    e = jnp.where(mask, jnp.exp(logit - m), 0.0)
    out_ref[...] = e / jnp.sum(e, axis=1, keepdims=True)


def _tc_last(s0, s1, t, dis, w, b):
    return pl.pallas_call(
        _tc4_body,
        grid=(_GRID,),
        in_specs=[_ROW, _ROW, _ROW, _COL1,
                  pl.BlockSpec((F, D3), lambda i: (0, 0)),
                  pl.BlockSpec((1, D3), lambda i: (0, 0))],
        out_specs=pl.BlockSpec((_BLK, D3), lambda i: (i, 0)),
        out_shape=jax.ShapeDtypeStruct((NPAD, D3), jnp.float32),
    )(s0, s1, t, dis, w, b)


@jax.jit
def kernel(x, edge_index, W1, b1, W2, b2, W3, b3):
    ei = edge_index.astype(jnp.int32)
    pad = jnp.full((EPAD - E,), N, jnp.int32)
    src2d = jnp.concatenate([ei[0], pad]).reshape(EPAD // CW, CW)
    dst2d = jnp.concatenate([ei[1], pad]).reshape(EPAD // CW, CW)
    x_pad = jnp.pad(x, ((0, NPAD - N), (0, 0)))
    w3p = jnp.pad(W3, ((0, 0), (0, D3 - NCLS)))
    b1r = b1.reshape(1, F)
    b2r = b2.reshape(1, F)
    b3r = jnp.pad(b3, (0, D3 - NCLS)).reshape(1, D3)
    zeros = jnp.zeros((ORS, F), jnp.float32)
    ones = jnp.ones((CW, F), jnp.float32)

    h1 = _tc_matmul(x_pad, W1)
    cnt = _count(ones, src2d, dst2d, zeros)
    dis, g1 = _tc_first(h1, cnt[:NPAD, 0:1], cnt[NPAD:, 0:1])

    s1 = _scatter(g1, src2d, dst2d, zeros)
    g2 = _tc_mid(s1[:NPAD], s1[NPAD:], g1, dis, b1r, W2, matmul=True)

    s2 = _scatter(g2, src2d, dst2d, zeros)
    t3 = _tc_mid(s2[:NPAD], s2[NPAD:], g2, dis, b2r, W2, matmul=False)

    s3 = _scatter(t3, src2d, dst2d, zeros)
    probs = _tc_last(s3[:NPAD], s3[NPAD:], t3, dis, w3p, b3r)
    return probs[:N, :NCLS]


# speedup vs baseline: 1.0809x; 1.0226x over previous
import functools

import jax
import jax.numpy as jnp
from jax import lax
from jax.experimental import pallas as pl
from jax.experimental.pallas import tpu as pltpu
from jax.experimental.pallas import tpu_sc as plsc

N = 10000
NPAD = 10240
F = 128
D3 = 64
NCLS = 40
E = 320000
C = 128
EPAD = 327680
KCH = EPAD // (32 * C)
HN = NPAD // 2
ORS = HN // 16

_MESH = dict(core_axis_name="c", subcore_axis_name="s")

CW = 64
NB = 4
NPH = 4


def _make_edge_kernel(with_gather, CW=CW, NPH=NPH):
    HK = EPAD // (32 * CW * NPH)

    @functools.partial(
        pl.kernel,
        out_type=jax.ShapeDtypeStruct((2 * NPAD, F), jnp.float32),
        mesh=plsc.VectorSubcoreMesh(**_MESH),
        scratch_types=[
            pltpu.VMEM((HK, CW), jnp.int32),
            pltpu.VMEM((HK, CW), jnp.int32),
            pltpu.VMEM((1, CW), jnp.int32),
            pltpu.VMEM((1, CW), jnp.int32),
            [pltpu.VMEM((CW, F), jnp.float32) for _ in range(NB)],
            pltpu.VMEM_SHARED((HN, F), jnp.float32),
            pltpu.VMEM_SHARED((HN, F), jnp.float32),
            [pltpu.SemaphoreType.DMA for _ in range(NB)],
        ],
    )
    def edge_kernel(*refs):
        (g_hbm, src_hbm, dst_hbm, zeros_hbm, out_hbm, src_v, dst_v,
         dsta_v, dstb_v, rows, acca, accb, sems) = refs
        cid = lax.axis_index("c")
        sid = lax.axis_index("s")
        wid = cid * 16 + sid

        def split_scatter(j, rows_v):
            for k in range(CW // 16):
                d = dst_v[j, pl.ds(k * 16, 16)]
                lo = d < HN
                dsta_v[0, pl.ds(k * 16, 16)] = jnp.where(lo, d, -1)
                dstb_v[0, pl.ds(k * 16, 16)] = jnp.where(lo, -1, d - HN)
            pltpu.sync_copy(
                rows_v,
                acca.at[plsc.Indices(dsta_v.at[0], ignored_value=-1)],
                add=True)
            pltpu.sync_copy(
                rows_v,
                accb.at[plsc.Indices(dstb_v.at[0], ignored_value=-1)],
                add=True)

        if not with_gather:
            pltpu.sync_copy(g_hbm.at[pl.ds(0, CW)], rows[0])
        pltpu.sync_copy(zeros_hbm, acca.at[pl.ds(sid * ORS, ORS)])
        pltpu.sync_copy(zeros_hbm, accb.at[pl.ds(sid * ORS, ORS)])
        plsc.subcore_barrier()

        for ph in range(NPH):
            base = wid * NPH * HK + ph * HK
            if with_gather:
                pltpu.sync_copy(src_hbm.at[pl.ds(base, HK)], src_v)
            pltpu.sync_copy(dst_hbm.at[pl.ds(base, HK)], dst_v)

            if with_gather:
                def gidx(j):
                    return src_v.at[j]

                for t in range(NB):
                    pltpu.async_copy(g_hbm.at[gidx(t)], rows[t], sems[t])

                def body(jj, carry):
                    for t in range(NB):
                        j = NB * jj + t
                        pltpu.make_async_copy(
                            g_hbm.at[gidx(j)], rows[t], sems[t]).wait()
                        split_scatter(j, rows[t])

                        @pl.when(j + NB < HK)
                        def _prefetch():
                            pltpu.async_copy(
                                g_hbm.at[gidx(j + NB)], rows[t], sems[t])
                    return carry
                lax.fori_loop(0, HK // NB, body, 0)
            else:
                def body(j, carry):
                    split_scatter(j, rows[0])
                    return carry
                lax.fori_loop(0, HK, body, 0)

        plsc.subcore_barrier()
        pltpu.sync_copy(acca.at[pl.ds(sid * ORS, ORS)],
                        out_hbm.at[pl.ds(cid * NPAD + sid * ORS, ORS)])
        pltpu.sync_copy(accb.at[pl.ds(sid * ORS, ORS)],
                        out_hbm.at[pl.ds(cid * NPAD + HN + sid * ORS, ORS)])

    return edge_kernel


_scatter = _make_edge_kernel(True)
_count = _make_edge_kernel(False, CW=128, NPH=2)


_BLK = 512
_GRID = NPAD // _BLK
_ROW = pl.BlockSpec((_BLK, F), lambda i: (i, 0))
_COL1 = pl.BlockSpec((_BLK, 1), lambda i: (i, 0))


def _dot(a, b):
    return lax.dot_general(a, b, (((1,), (0,)), ((), ())),
                           precision=lax.Precision.HIGHEST,
                           preferred_element_type=jnp.float32)


def _tcmm_body(x_ref, w_ref, h_ref):
    h_ref[...] = _dot(x_ref[...], w_ref[...])


def _tc_matmul(x, w):
    return pl.pallas_call(
        _tcmm_body,
        grid=(_GRID,),
        in_specs=[_ROW, pl.BlockSpec((F, F), lambda i: (0, 0))],
        out_specs=_ROW,
        out_shape=jax.ShapeDtypeStruct((NPAD, F), jnp.float32),
    )(x, w)


def _tc1_body(h_ref, c0_ref, c1_ref, dis_ref, g_ref):
    deg = 1.0 + c0_ref[...] + c1_ref[...]
    dis = lax.rsqrt(deg)
    dis_ref[...] = dis
    g_ref[...] = dis * h_ref[...]


def _tc_first(h, c0, c1):
    return pl.pallas_call(
        _tc1_body,
        grid=(_GRID,),
        in_specs=[_ROW, _COL1, _COL1],
        out_specs=[_COL1, _ROW],
        out_shape=[
            jax.ShapeDtypeStruct((NPAD, 1), jnp.float32),
            jax.ShapeDtypeStruct((NPAD, F), jnp.float32),
        ],
    )(h, c0, c1)


def _tcmid_body(matmul, s0, s1, g, dis_ref, b_ref, w_ref, out_ref):
    dis = dis_ref[...]
    h = dis * (s0[...] + s1[...] + g[...]) + b_ref[...]
    h = jnp.maximum(h, 0.0)
    if matmul:
        out_ref[...] = dis * _dot(h, w_ref[...])
    else:
        out_ref[...] = dis * h


def _tc_mid(s0, s1, g, dis, b, w, matmul):
    return pl.pallas_call(
        functools.partial(_tcmid_body, matmul),
        grid=(_GRID,),
        in_specs=[_ROW, _ROW, _ROW, _COL1,
                  pl.BlockSpec((1, F), lambda i: (0, 0)),
                  pl.BlockSpec((F, F), lambda i: (0, 0))],
        out_specs=_ROW,
        out_shape=jax.ShapeDtypeStruct((NPAD, F), jnp.float32),
    )(s0, s1, g, dis, b, w)


def _tc4_body(s0, s1, t_ref, dis_ref, w_ref, b_ref, out_ref):
    z = dis_ref[...] * (s0[...] + s1[...] + t_ref[...])
    logit = _dot(z, w_ref[...]) + b_ref[...]
    col = lax.broadcasted_iota(jnp.int32, logit.shape, 1)
    mask = col < NCLS
    logit = jnp.where(mask, logit, -jnp.inf)
    m = jnp.max(logit, axis=1, keepdims=True)
    e = jnp.where(mask, jnp.exp(logit - m), 0.0)
    out_ref[...] = e / jnp.sum(e, axis=1, keepdims=True)


def _tc_last(s0, s1, t, dis, w, b):
    return pl.pallas_call(
        _tc4_body,
        grid=(_GRID,),
        in_specs=[_ROW, _ROW, _ROW, _COL1,
                  pl.BlockSpec((F, D3), lambda i: (0, 0)),
                  pl.BlockSpec((1, D3), lambda i: (0, 0))],
        out_specs=pl.BlockSpec((_BLK, D3), lambda i: (i, 0)),
        out_shape=jax.ShapeDtypeStruct((NPAD, D3), jnp.float32),
    )(s0, s1, t, dis, w, b)


@jax.jit
def kernel(x, edge_index, W1, b1, W2, b2, W3, b3):
    ei = edge_index.astype(jnp.int32)
    pad = jnp.full((EPAD - E,), N, jnp.int32)
    src2d = jnp.concatenate([ei[0], pad]).reshape(EPAD // CW, CW)
    dst2d = jnp.concatenate([ei[1], pad]).reshape(EPAD // CW, CW)
    x_pad = jnp.pad(x, ((0, NPAD - N), (0, 0)))
    w3p = jnp.pad(W3, ((0, 0), (0, D3 - NCLS)))
    b1r = b1.reshape(1, F)
    b2r = b2.reshape(1, F)
    b3r = jnp.pad(b3, (0, D3 - NCLS)).reshape(1, D3)
    zeros = jnp.zeros((ORS, F), jnp.float32)
    ones = jnp.ones((128, F), jnp.float32)
    dst128 = dst2d.reshape(EPAD // 128, 128)

    h1 = _tc_matmul(x_pad, W1)
    cnt = _count(ones, dst128, dst128, zeros)
    dis, g1 = _tc_first(h1, cnt[:NPAD, 0:1], cnt[NPAD:, 0:1])

    s1 = _scatter(g1, src2d, dst2d, zeros)
    g2 = _tc_mid(s1[:NPAD], s1[NPAD:], g1, dis, b1r, W2, matmul=True)

    s2 = _scatter(g2, src2d, dst2d, zeros)
    t3 = _tc_mid(s2[:NPAD], s2[NPAD:], g2, dis, b2r, W2, matmul=False)

    s3 = _scatter(t3, src2d, dst2d, zeros)
    probs = _tc_last(s3[:NPAD], s3[NPAD:], t3, dis, w3p, b3r)
    return probs[:N, :NCLS]
